# Initial kernel scaffold; baseline (speedup 1.0000x reference)
import jax
import jax.numpy as jnp
from jax.experimental import pallas as pl


def kernel(x, batch, cheb_w, cheb_b, fc_w, fc_b):
    raise NotImplementedError("write your pallas kernel here")


# speedup vs baseline: 2251.9810x; 2251.9810x over previous
import jax
import jax.numpy as jnp
from jax.experimental import pallas as pl
from jax.experimental.pallas import tpu as pltpu

_HIGHEST = jax.lax.Precision.HIGHEST
_R = 256


def _dot(a, b, dims):
    return jax.lax.dot_general(
        a, b, dimension_numbers=(dims, ((), ())),
        precision=_HIGHEST, preferred_element_type=jnp.float32)


def _rgcnn_body(x_ref, cw_ref, cb_ref, fw_ref, fb_ref, o_ref,
                a_ref, deg_ref, p_ref):
    x = x_ref[...]
    n = x.shape[0]
    nblk = n // _R

    y = x * x
    sq = jnp.sum(y, axis=1, keepdims=True)
    ones_row = jnp.ones((1, x.shape[1]), jnp.float32)
    sq_row = _dot(ones_row, y, ((1,), (1,)))

    def build_blk(i, _):
        r0 = i * _R
        xb = x_ref[pl.ds(r0, _R), :]
        sqb = jnp.sum(xb * xb, axis=1, keepdims=True)
        gb = jax.lax.dot_general(
            xb, x, dimension_numbers=((((1,), (1,))), ((), ())),
            preferred_element_type=jnp.float32)
        d2b = sqb + sq_row - 2.0 * gb
        ii = jax.lax.broadcasted_iota(jnp.int32, (_R, n), 0) + r0
        jj = jax.lax.broadcasted_iota(jnp.int32, (_R, n), 1)
        ab = jnp.where(ii == jj, 0.0, jnp.exp(-d2b))
        a_ref[pl.ds(r0, _R), :] = ab
        deg_ref[pl.ds(r0, _R), :] = jnp.sum(ab, axis=1, keepdims=True)
        return 0

    jax.lax.fori_loop(0, nblk, build_blk, 0)
    deg = deg_ref[...]
    dis = jnp.where(deg > 0.0, jax.lax.rsqrt(deg), 0.0)

    def prop(h):
        dish = dis * h

        def prop_blk(i, _):
            r0 = i * _R
            ablk = a_ref[pl.ds(r0, _R), :]
            p_ref[pl.ds(r0, _R), :] = _dot(ablk, dish, ((1,), (0,)))
            return 0

        jax.lax.fori_loop(0, nblk, prop_blk, 0)
        return -dis * p_ref[...]

    k_total = cw_ref.shape[0]
    tx0 = x
    out = _dot(tx0, cw_ref[0], ((1,), (0,)))
    tx1 = prop(tx0)
    out = out + _dot(tx1, cw_ref[1], ((1,), (0,)))
    for k in range(2, k_total):
        tx2 = 2.0 * prop(tx1) - tx0
        out = out + _dot(tx2, cw_ref[k], ((1,), (0,)))
        tx0, tx1 = tx1, tx2

    out = jnp.maximum(out + cb_ref[...], 0.0)
    pooled = jnp.max(out, axis=0, keepdims=True)
    o_ref[...] = _dot(pooled, fw_ref[...], ((1,), (0,))) + fb_ref[...]


def kernel(x, batch, cheb_w, cheb_b, fc_w, fc_b):
    del batch
    n, f = x.shape
    return pl.pallas_call(
        _rgcnn_body,
        out_shape=jax.ShapeDtypeStruct((1, fc_w.shape[1]), jnp.float32),
        scratch_shapes=[
            pltpu.VMEM((n, n), jnp.float32),
            pltpu.VMEM((n, 1), jnp.float32),
            pltpu.VMEM((n, f), jnp.float32),
        ],
        compiler_params=pltpu.CompilerParams(
            vmem_limit_bytes=60 * 1024 * 1024),
    )(x, cheb_w, cheb_b.reshape(1, -1), fc_w, fc_b.reshape(1, -1))


# speedup vs baseline: 5510.0093x; 2.4467x over previous
import jax
import jax.numpy as jnp
from jax.experimental import pallas as pl
from jax.experimental.pallas import tpu as pltpu

_HIGHEST = jax.lax.Precision.HIGHEST
_R = 256


def _dot(a, b, dims):
    return jax.lax.dot_general(
        a, b, dimension_numbers=(dims, ((), ())),
        precision=_HIGHEST, preferred_element_type=jnp.float32)


def _rgcnn_body(x_ref, cw_ref, cb_ref, fw_ref, fb_ref, o_ref,
                a_ref, deg_ref, p_ref):
    x = x_ref[...]
    n = x.shape[0]
    nblk = n // _R

    y = x * x
    sq = jnp.sum(y, axis=1, keepdims=True)
    ones_row = jnp.ones((1, x.shape[1]), jnp.float32)
    sq_row = _dot(ones_row, y, ((1,), (1,)))

    def build_blk(i, _):
        r0 = i * _R
        xb = x_ref[pl.ds(r0, _R), :]
        sqb = jnp.sum(xb * xb, axis=1, keepdims=True)
        gb = jax.lax.dot_general(
            xb, x, dimension_numbers=((((1,), (1,))), ((), ())),
            preferred_element_type=jnp.float32)
        d2b = sqb + sq_row - 2.0 * gb
        ii = jax.lax.broadcasted_iota(jnp.int32, (_R, n), 0) + r0
        jj = jax.lax.broadcasted_iota(jnp.int32, (_R, n), 1)
        ab = jnp.where(ii == jj, 0.0, jnp.exp(-d2b))
        a_ref[pl.ds(r0, _R), :] = ab.astype(jnp.bfloat16)
        deg_ref[pl.ds(r0, _R), :] = jnp.sum(ab, axis=1, keepdims=True)
        return 0

    jax.lax.fori_loop(0, nblk, build_blk, 0)
    deg = deg_ref[...]
    dis = jnp.where(deg > 0.0, jax.lax.rsqrt(deg), 0.0)

    def prop(h):
        dish = (dis * h).astype(jnp.bfloat16)

        def prop_blk(i, _):
            r0 = i * _R
            ablk = a_ref[pl.ds(r0, _R), :]
            p_ref[pl.ds(r0, _R), :] = jax.lax.dot_general(
                ablk, dish, dimension_numbers=(((1,), (0,)), ((), ())),
                preferred_element_type=jnp.float32)
            return 0

        jax.lax.fori_loop(0, nblk, prop_blk, 0)
        return -dis * p_ref[...]

    k_total = cw_ref.shape[0]
    tx0 = x
    out = _dot(tx0, cw_ref[0], ((1,), (0,)))
    tx1 = prop(tx0)
    out = out + _dot(tx1, cw_ref[1], ((1,), (0,)))
    for k in range(2, k_total):
        tx2 = 2.0 * prop(tx1) - tx0
        out = out + _dot(tx2, cw_ref[k], ((1,), (0,)))
        tx0, tx1 = tx1, tx2

    out = jnp.maximum(out + cb_ref[...], 0.0)
    pooled = jnp.max(out, axis=0, keepdims=True)
    o_ref[...] = _dot(pooled, fw_ref[...], ((1,), (0,))) + fb_ref[...]


def kernel(x, batch, cheb_w, cheb_b, fc_w, fc_b):
    del batch
    n, f = x.shape
    return pl.pallas_call(
        _rgcnn_body,
        out_shape=jax.ShapeDtypeStruct((1, fc_w.shape[1]), jnp.float32),
        scratch_shapes=[
            pltpu.VMEM((n, n), jnp.bfloat16),
            pltpu.VMEM((n, 1), jnp.float32),
            pltpu.VMEM((n, f), jnp.float32),
        ],
        compiler_params=pltpu.CompilerParams(
            vmem_limit_bytes=60 * 1024 * 1024),
    )(x, cheb_w, cheb_b.reshape(1, -1), fc_w, fc_b.reshape(1, -1))


# speedup vs baseline: 10176.0765x; 1.8468x over previous
import jax
import jax.numpy as jnp
from jax.experimental import pallas as pl
from jax.experimental.pallas import tpu as pltpu

_R = 256


def _dot(a, b, dims, prec=None):
    return jax.lax.dot_general(
        a, b, dimension_numbers=(dims, ((), ())),
        precision=prec, preferred_element_type=jnp.float32)


def _rgcnn_body(x_ref, xt_ref, cwt_ref, cb_ref, fw_ref, fb_ref, o_ref, a_ref):
    x = x_ref[...]
    n = x.shape[0]
    nblk = n // _R

    ones_row = jnp.ones((1, x.shape[1]), jnp.float32)
    sq_row = _dot(ones_row, x * x, ((1,), (1,)))

    def build_blk(i, _):
        r0 = i * _R
        xb = x_ref[pl.ds(r0, _R), :]
        sqb = jnp.sum(xb * xb, axis=1, keepdims=True)
        gb = _dot(xb, x, ((1,), (1,)))
        d2b = sqb + sq_row - 2.0 * gb
        ii = jax.lax.broadcasted_iota(jnp.int32, (_R, n), 0) + r0
        jj = jax.lax.broadcasted_iota(jnp.int32, (_R, n), 1)
        ab = jnp.where(ii == jj, 0.0, jnp.exp(-d2b))
        a_ref[pl.ds(r0, _R), :] = ab.astype(jnp.bfloat16)
        return 0

    jax.lax.fori_loop(0, nblk, build_blk, 0)
    a = a_ref[...]

    ones_n = jnp.ones((1, n), jnp.bfloat16)
    deg = _dot(ones_n, a, ((1,), (0,)))
    dis = jnp.where(deg > 0.0, jax.lax.rsqrt(deg), 0.0)

    def prop(ht):
        hd = (dis * ht).astype(jnp.bfloat16)
        return -dis * _dot(hd, a, ((1,), (0,)))

    k_total = cwt_ref.shape[0]
    tx0 = xt_ref[...]
    outt = _dot(cwt_ref[0], tx0, ((1,), (0,)))
    tx1 = prop(tx0)
    outt = outt + _dot(cwt_ref[1], tx1, ((1,), (0,)))
    for k in range(2, k_total):
        tx2 = 2.0 * prop(tx1) - tx0
        outt = outt + _dot(cwt_ref[k], tx2, ((1,), (0,)))
        tx0, tx1 = tx1, tx2

    outt = jnp.maximum(outt + cb_ref[...], 0.0)
    pooled = jnp.max(outt, axis=1, keepdims=True)
    o_ref[...] = _dot(pooled, fw_ref[...], ((0,), (0,))) + fb_ref[...]


def kernel(x, batch, cheb_w, cheb_b, fc_w, fc_b):
    del batch
    n = x.shape[0]
    return pl.pallas_call(
        _rgcnn_body,
        out_shape=jax.ShapeDtypeStruct((1, fc_w.shape[1]), jnp.float32),
        scratch_shapes=[pltpu.VMEM((n, n), jnp.bfloat16)],
        compiler_params=pltpu.CompilerParams(
            vmem_limit_bytes=60 * 1024 * 1024),
    )(x, x.T, jnp.transpose(cheb_w, (0, 2, 1)), cheb_b.reshape(-1, 1),
      fc_w, fc_b.reshape(1, -1))


# speedup vs baseline: 10395.1741x; 1.0215x over previous
import jax
import jax.numpy as jnp
from jax.experimental import pallas as pl
from jax.experimental.pallas import tpu as pltpu

_R = 256


def _dot(a, b, dims, prec=None):
    return jax.lax.dot_general(
        a, b, dimension_numbers=(dims, ((), ())),
        precision=prec, preferred_element_type=jnp.float32)


def _rgcnn_body(x_ref, xt_ref, cwt_ref, cb_ref, fw_ref, fb_ref, o_ref, a_ref):
    x = x_ref[...]
    n = x.shape[0]
    nblk = n // _R

    ones_row = jnp.ones((1, x.shape[1]), jnp.float32)
    sq_row = _dot(ones_row, x * x, ((1,), (1,)))

    def build_blk(i, deg_acc):
        r0 = i * _R
        xb = x_ref[pl.ds(r0, _R), :]
        sqb = jnp.sum(xb * xb, axis=1, keepdims=True)
        gb = _dot(xb, x, ((1,), (1,)))
        d2b = sqb + sq_row - 2.0 * gb
        ii = jax.lax.broadcasted_iota(jnp.int32, (_R, n), 0) + r0
        jj = jax.lax.broadcasted_iota(jnp.int32, (_R, n), 1)
        ab = jnp.where(ii == jj, 0.0, jnp.exp(-d2b))
        a_ref[pl.ds(r0, _R), :] = ab.astype(jnp.bfloat16)
        return deg_acc + jnp.sum(ab, axis=0, keepdims=True)

    deg = jax.lax.fori_loop(0, nblk, build_blk,
                            jnp.zeros((1, n), jnp.float32))
    a = a_ref[...]
    dis = jnp.where(deg > 0.0, jax.lax.rsqrt(deg), 0.0)

    def prop(ht):
        hd = (dis * ht).astype(jnp.bfloat16)
        return -dis * _dot(hd, a, ((1,), (0,)))

    k_total = cwt_ref.shape[0]
    tx0 = xt_ref[...]
    outt = _dot(cwt_ref[0], tx0, ((1,), (0,)))
    tx1 = prop(tx0)
    outt = outt + _dot(cwt_ref[1], tx1, ((1,), (0,)))
    for k in range(2, k_total):
        tx2 = 2.0 * prop(tx1) - tx0
        outt = outt + _dot(cwt_ref[k], tx2, ((1,), (0,)))
        tx0, tx1 = tx1, tx2

    outt = jnp.maximum(outt + cb_ref[...], 0.0)
    pooled = jnp.max(outt, axis=1, keepdims=True)
    o_ref[...] = _dot(pooled, fw_ref[...], ((0,), (0,))) + fb_ref[...]


def kernel(x, batch, cheb_w, cheb_b, fc_w, fc_b):
    del batch
    n = x.shape[0]
    return pl.pallas_call(
        _rgcnn_body,
        out_shape=jax.ShapeDtypeStruct((1, fc_w.shape[1]), jnp.float32),
        scratch_shapes=[pltpu.VMEM((n, n), jnp.bfloat16)],
        compiler_params=pltpu.CompilerParams(
            vmem_limit_bytes=60 * 1024 * 1024),
    )(x, x.T, jnp.transpose(cheb_w, (0, 2, 1)), cheb_b.reshape(-1, 1),
      fc_w, fc_b.reshape(1, -1))


# speedup vs baseline: 11887.5937x; 1.1436x over previous
import jax
import jax.numpy as jnp
from jax.experimental import pallas as pl
from jax.experimental.pallas import tpu as pltpu

_R = 512


def _dot(a, b, dims, prec=None):
    return jax.lax.dot_general(
        a, b, dimension_numbers=(dims, ((), ())),
        precision=prec, preferred_element_type=jnp.float32)


def _rgcnn_body(x_ref, xt_ref, cwt_ref, cb_ref, fw_ref, fb_ref, o_ref, a_ref):
    xt = xt_ref[...]
    n = xt.shape[1]
    nblk = n // _R

    sq_row = jnp.sum(xt * xt, axis=0, keepdims=True)

    def build_blk(i, deg_acc):
        r0 = i * _R
        xb = x_ref[pl.ds(r0, _R), :]
        sqb = jnp.sum(xb * xb, axis=1, keepdims=True)
        gb2 = _dot(2.0 * xb, xt, ((1,), (0,)))
        ii = jax.lax.broadcasted_iota(jnp.int32, (_R, n), 0) + r0
        jj = jax.lax.broadcasted_iota(jnp.int32, (_R, n), 1)
        ab = jnp.where(ii == jj, 0.0, jnp.exp(gb2 - (sqb + sq_row)))
        a_ref[pl.ds(r0, _R), :] = ab.astype(jnp.bfloat16)
        return deg_acc + jnp.sum(ab, axis=0, keepdims=True)

    deg = jax.lax.fori_loop(0, nblk, build_blk,
                            jnp.zeros((1, n), jnp.float32))
    a = a_ref[...]
    dis = jnp.where(deg > 0.0, jax.lax.rsqrt(deg), 0.0)

    def prop(ht):
        hd = (dis * ht).astype(jnp.bfloat16)
        return -dis * _dot(hd, a, ((1,), (0,)))

    k_total = cwt_ref.shape[0]
    tx0 = xt_ref[...]
    outt = _dot(cwt_ref[0], tx0, ((1,), (0,)))
    tx1 = prop(tx0)
    outt = outt + _dot(cwt_ref[1], tx1, ((1,), (0,)))
    for k in range(2, k_total):
        tx2 = 2.0 * prop(tx1) - tx0
        outt = outt + _dot(cwt_ref[k], tx2, ((1,), (0,)))
        tx0, tx1 = tx1, tx2

    outt = jnp.maximum(outt + cb_ref[...], 0.0)
    pooled = jnp.max(outt, axis=1, keepdims=True)
    o_ref[...] = _dot(pooled, fw_ref[...], ((0,), (0,))) + fb_ref[...]


def kernel(x, batch, cheb_w, cheb_b, fc_w, fc_b):
    del batch
    n = x.shape[0]
    return pl.pallas_call(
        _rgcnn_body,
        out_shape=jax.ShapeDtypeStruct((1, fc_w.shape[1]), jnp.float32),
        scratch_shapes=[pltpu.VMEM((n, n), jnp.bfloat16)],
        compiler_params=pltpu.CompilerParams(
            vmem_limit_bytes=60 * 1024 * 1024),
    )(x, x.T, jnp.transpose(cheb_w, (0, 2, 1)), cheb_b.reshape(-1, 1),
      fc_w, fc_b.reshape(1, -1))


# speedup vs baseline: 12544.3816x; 1.0552x over previous
import jax
import jax.numpy as jnp
from jax.experimental import pallas as pl
from jax.experimental.pallas import tpu as pltpu

_R = 512


def _dot(a, b, dims, prec=None):
    return jax.lax.dot_general(
        a, b, dimension_numbers=(dims, ((), ())),
        precision=prec, preferred_element_type=jnp.float32)


def _rgcnn_body(x_ref, xt_ref, cwt_ref, cb_ref, fw_ref, fb_ref, o_ref, a_ref):
    xt = xt_ref[...]
    n = xt.shape[1]
    nblk = n // _R

    sq_row = jnp.sum(xt * xt, axis=0, keepdims=True)

    y16 = xt.astype(jnp.bfloat16).astype(jnp.float32)
    adiag = jnp.exp(2.0 * jnp.sum(y16 * y16, axis=0, keepdims=True)
                    - 2.0 * sq_row)

    def build_blk(i, deg_acc):
        r0 = i * _R
        xb = x_ref[pl.ds(r0, _R), :]
        sqb = jnp.sum(xb * xb, axis=1, keepdims=True)
        gb2 = _dot(2.0 * xb, xt, ((1,), (0,)))
        ab = jnp.exp(gb2 - (sqb + sq_row))
        a_ref[pl.ds(r0, _R), :] = ab.astype(jnp.bfloat16)
        return deg_acc + jnp.sum(ab, axis=0, keepdims=True)

    deg = jax.lax.fori_loop(0, nblk, build_blk,
                            jnp.zeros((1, n), jnp.float32))
    deg = deg - adiag
    a = a_ref[...]
    dis = jnp.where(deg > 0.0, jax.lax.rsqrt(deg), 0.0)

    def prop(ht):
        hdf = dis * ht
        hd = hdf.astype(jnp.bfloat16)
        p = _dot(hd, a, ((1,), (0,))) - hdf * adiag
        return -dis * p

    k_total = cwt_ref.shape[0]
    tx0 = xt_ref[...]
    outt = _dot(cwt_ref[0], tx0, ((1,), (0,)))
    tx1 = prop(tx0)
    outt = outt + _dot(cwt_ref[1], tx1, ((1,), (0,)))
    for k in range(2, k_total):
        tx2 = 2.0 * prop(tx1) - tx0
        outt = outt + _dot(cwt_ref[k], tx2, ((1,), (0,)))
        tx0, tx1 = tx1, tx2

    outt = jnp.maximum(outt + cb_ref[...], 0.0)
    pooled = jnp.max(outt, axis=1, keepdims=True)
    o_ref[...] = _dot(pooled, fw_ref[...], ((0,), (0,))) + fb_ref[...]


def kernel(x, batch, cheb_w, cheb_b, fc_w, fc_b):
    del batch
    n = x.shape[0]
    return pl.pallas_call(
        _rgcnn_body,
        out_shape=jax.ShapeDtypeStruct((1, fc_w.shape[1]), jnp.float32),
        scratch_shapes=[pltpu.VMEM((n, n), jnp.bfloat16)],
        compiler_params=pltpu.CompilerParams(
            vmem_limit_bytes=60 * 1024 * 1024),
    )(x, x.T, jnp.transpose(cheb_w, (0, 2, 1)), cheb_b.reshape(-1, 1),
      fc_w, fc_b.reshape(1, -1))


# speedup vs baseline: 12571.0843x; 1.0021x over previous
import jax
import jax.numpy as jnp
from jax.experimental import pallas as pl
from jax.experimental.pallas import tpu as pltpu

_R = 512


def _dot(a, b, dims, prec=None):
    return jax.lax.dot_general(
        a, b, dimension_numbers=(dims, ((), ())),
        precision=prec, preferred_element_type=jnp.float32)


def _rgcnn_body(x_ref, xt_ref, cwt_ref, cb_ref, fw_ref, fb_ref, o_ref, a_ref):
    xt = xt_ref[...]
    n = xt.shape[1]
    nblk = n // _R

    sq_row = jnp.sum(xt * xt, axis=0, keepdims=True)

    def build_blk(i, deg_acc):
        r0 = i * _R
        xb = x_ref[pl.ds(r0, _R), :]
        sqb = jnp.sum(xb * xb, axis=1, keepdims=True)
        gb2 = _dot(2.0 * xb, xt, ((1,), (0,)))
        ab = jnp.exp(gb2 - (sqb + sq_row))
        a_ref[pl.ds(r0, _R), :] = ab.astype(jnp.bfloat16)
        return deg_acc + jnp.sum(ab, axis=0, keepdims=True)

    degsum = jax.lax.fori_loop(0, nblk, build_blk,
                               jnp.zeros((1, n), jnp.float32))
    a = a_ref[...]

    eye = (jax.lax.broadcasted_iota(jnp.int32, (128, 128), 0) ==
           jax.lax.broadcasted_iota(jnp.int32, (128, 128), 1)
           ).astype(jnp.float32)
    adiag = jnp.concatenate(
        [jnp.sum(a_ref[c:c + 128, c:c + 128].astype(jnp.float32) * eye,
                 axis=0, keepdims=True)
         for c in range(0, n, 128)], axis=1)
    deg = degsum - adiag
    dis = jnp.where(deg > 0.0, jax.lax.rsqrt(deg), 0.0)

    def prop(ht):
        hd = (dis * ht).astype(jnp.bfloat16)
        hd32 = hd.astype(jnp.float32)
        p = _dot(hd, a, ((1,), (0,))) - hd32 * adiag
        return -dis * p

    k_total = cwt_ref.shape[0]
    tx0 = xt_ref[...]
    outt = _dot(cwt_ref[0], tx0, ((1,), (0,)))
    tx1 = prop(tx0)
    outt = outt + _dot(cwt_ref[1], tx1, ((1,), (0,)))
    for k in range(2, k_total):
        tx2 = 2.0 * prop(tx1) - tx0
        outt = outt + _dot(cwt_ref[k], tx2, ((1,), (0,)))
        tx0, tx1 = tx1, tx2

    outt = jnp.maximum(outt + cb_ref[...], 0.0)
    pooled = jnp.max(outt, axis=1, keepdims=True)
    o_ref[...] = _dot(pooled, fw_ref[...], ((0,), (0,))) + fb_ref[...]


def kernel(x, batch, cheb_w, cheb_b, fc_w, fc_b):
    del batch
    n = x.shape[0]
    return pl.pallas_call(
        _rgcnn_body,
        out_shape=jax.ShapeDtypeStruct((1, fc_w.shape[1]), jnp.float32),
        scratch_shapes=[pltpu.VMEM((n, n), jnp.bfloat16)],
        compiler_params=pltpu.CompilerParams(
            vmem_limit_bytes=60 * 1024 * 1024),
    )(x, x.T, jnp.transpose(cheb_w, (0, 2, 1)), cheb_b.reshape(-1, 1),
      fc_w, fc_b.reshape(1, -1))


# speedup vs baseline: 12807.8854x; 1.0188x over previous
import jax
import jax.numpy as jnp
from jax.experimental import pallas as pl
from jax.experimental.pallas import tpu as pltpu

_R = 512


def _dot(a, b, dims, prec=None):
    return jax.lax.dot_general(
        a, b, dimension_numbers=(dims, ((), ())),
        precision=prec, preferred_element_type=jnp.float32)


def _rgcnn_body(x_ref, xt_ref, cwt_ref, cb_ref, fw_ref, fb_ref, o_ref, a_ref):
    xt = xt_ref[...]
    n = xt.shape[1]
    nblk = n // _R

    sq_row = jnp.sum(xt * xt, axis=0, keepdims=True)

    def build_blk(i, deg_acc):
        r0 = i * _R
        xb = x_ref[pl.ds(r0, _R), :]
        sqb = jnp.sum(xb * xb, axis=1, keepdims=True)
        gb2 = _dot(2.0 * xb, xt, ((1,), (0,)))
        ab = jnp.exp(gb2 - (sqb + sq_row))
        a_ref[pl.ds(r0, _R), :] = ab.astype(jnp.bfloat16)
        return deg_acc + jnp.sum(ab, axis=0, keepdims=True)

    degsum = jnp.zeros((1, n), jnp.float32)
    for i in range(nblk):
        degsum = build_blk(i, degsum)
    a = a_ref[...]

    eye = (jax.lax.broadcasted_iota(jnp.int32, (128, 128), 0) ==
           jax.lax.broadcasted_iota(jnp.int32, (128, 128), 1)
           ).astype(jnp.float32)
    adiag = jnp.concatenate(
        [jnp.sum(a_ref[c:c + 128, c:c + 128].astype(jnp.float32) * eye,
                 axis=0, keepdims=True)
         for c in range(0, n, 128)], axis=1)
    deg = degsum - adiag
    dis = jnp.where(deg > 0.0, jax.lax.rsqrt(deg), 0.0)

    def prop(ht):
        hd = (dis * ht).astype(jnp.bfloat16)
        hd32 = hd.astype(jnp.float32)
        p = _dot(hd, a, ((1,), (0,))) - hd32 * adiag
        return -dis * p

    k_total = cwt_ref.shape[0]
    tx0 = xt_ref[...]
    outt = _dot(cwt_ref[0], tx0, ((1,), (0,)))
    tx1 = prop(tx0)
    outt = outt + _dot(cwt_ref[1], tx1, ((1,), (0,)))
    for k in range(2, k_total):
        tx2 = 2.0 * prop(tx1) - tx0
        outt = outt + _dot(cwt_ref[k], tx2, ((1,), (0,)))
        tx0, tx1 = tx1, tx2

    outt = jnp.maximum(outt + cb_ref[...], 0.0)
    pooled = jnp.max(outt, axis=1, keepdims=True)
    o_ref[...] = _dot(pooled, fw_ref[...], ((0,), (0,))) + fb_ref[...]


def kernel(x, batch, cheb_w, cheb_b, fc_w, fc_b):
    del batch
    n = x.shape[0]
    return pl.pallas_call(
        _rgcnn_body,
        out_shape=jax.ShapeDtypeStruct((1, fc_w.shape[1]), jnp.float32),
        scratch_shapes=[pltpu.VMEM((n, n), jnp.bfloat16)],
        compiler_params=pltpu.CompilerParams(
            vmem_limit_bytes=60 * 1024 * 1024),
    )(x, x.T, jnp.transpose(cheb_w, (0, 2, 1)), cheb_b.reshape(-1, 1),
      fc_w, fc_b.reshape(1, -1))


# speedup vs baseline: 13693.2495x; 1.0691x over previous
import jax
import jax.numpy as jnp
from jax.experimental import pallas as pl
from jax.experimental.pallas import tpu as pltpu

_HIGHEST = jax.lax.Precision.HIGHEST
_R = 512


def _dot(a, b, dims, prec=None):
    return jax.lax.dot_general(
        a, b, dimension_numbers=(dims, ((), ())),
        precision=prec, preferred_element_type=jnp.float32)


def _rgcnn_body(x_ref, cw_ref, cb_ref, fw_ref, fb_ref, o_ref, a_ref):
    x = x_ref[...]
    n, f = x.shape
    nblk = n // _R

    eye_f = (jax.lax.broadcasted_iota(jnp.int32, (f, f), 0) ==
             jax.lax.broadcasted_iota(jnp.int32, (f, f), 1)
             ).astype(jnp.float32)
    xt = _dot(eye_f, x, ((1,), (1,)), _HIGHEST)
    sq_row = jnp.sum(xt * xt, axis=0, keepdims=True)

    def build_blk(i, deg_acc):
        r0 = i * _R
        xb = x_ref[pl.ds(r0, _R), :]
        sqb = jnp.sum(xb * xb, axis=1, keepdims=True)
        gb2 = _dot(2.0 * xb, xt, ((1,), (0,)))
        ab = jnp.exp(gb2 - (sqb + sq_row))
        a_ref[pl.ds(r0, _R), :] = ab.astype(jnp.bfloat16)
        return deg_acc + jnp.sum(ab, axis=0, keepdims=True)

    degsum = jnp.zeros((1, n), jnp.float32)
    for i in range(nblk):
        degsum = build_blk(i, degsum)
    a = a_ref[...]

    eye = (jax.lax.broadcasted_iota(jnp.int32, (128, 128), 0) ==
           jax.lax.broadcasted_iota(jnp.int32, (128, 128), 1)
           ).astype(jnp.float32)
    adiag = jnp.concatenate(
        [jnp.sum(a_ref[c:c + 128, c:c + 128].astype(jnp.float32) * eye,
                 axis=0, keepdims=True)
         for c in range(0, n, 128)], axis=1)
    deg = degsum - adiag
    dis = jnp.where(deg > 0.0, jax.lax.rsqrt(deg), 0.0)

    def prop(ht):
        hd = (dis * ht).astype(jnp.bfloat16)
        hd32 = hd.astype(jnp.float32)
        p = _dot(hd, a, ((1,), (0,))) - hd32 * adiag
        return -dis * p

    k_total = cw_ref.shape[0]
    tx0 = xt

    def chan(k, tx):
        return _dot(cw_ref[k], tx, ((0,), (0,)))

    outt = chan(0, tx0)
    tx1 = prop(tx0)
    outt = outt + chan(1, tx1)
    for k in range(2, k_total):
        tx2 = 2.0 * prop(tx1) - tx0
        outt = outt + chan(k, tx2)
        tx0, tx1 = tx1, tx2

    cb_col = _dot(eye, cb_ref[...].reshape(1, -1), ((1,), (1,)),
                  _HIGHEST)
    outt = jnp.maximum(outt + cb_col, 0.0)
    pooled = jnp.max(outt, axis=1, keepdims=True)
    o_ref[...] = _dot(pooled, fw_ref[...], ((0,), (0,))) + fb_ref[...].reshape(1, -1)


def kernel(x, batch, cheb_w, cheb_b, fc_w, fc_b):
    del batch
    n = x.shape[0]
    return pl.pallas_call(
        _rgcnn_body,
        out_shape=jax.ShapeDtypeStruct((1, fc_w.shape[1]), jnp.float32),
        scratch_shapes=[pltpu.VMEM((n, n), jnp.bfloat16)],
        compiler_params=pltpu.CompilerParams(
            vmem_limit_bytes=60 * 1024 * 1024),
    )(x, cheb_w, cheb_b, fc_w, fc_b)
